# baseline (speedup 1.0000x reference)
import functools

import jax
import jax.numpy as jnp
from jax import lax
from jax.experimental import pallas as pl
from jax.experimental.pallas import tpu as pltpu
from jax.experimental.pallas import tpu_sc as plsc

C_LEN = 512
N_POOL = 128
TOP_K = 16
POOL_Q = 4096

NUM_CORES = 2
NUM_SUBCORES = 16
NUM_WORKERS = NUM_CORES * NUM_SUBCORES
CHUNK = 128


def _pool_body(x_ref, w_ref, out_ref):
    x = x_ref[0]
    w = w_ref[...]
    h1 = x.astype(jnp.bfloat16)
    r1 = x - h1.astype(jnp.float32)
    h2 = r1.astype(jnp.bfloat16)
    r2 = r1 - h2.astype(jnp.float32)
    h3 = r2.astype(jnp.bfloat16)
    dot = lambda h: jnp.dot(h, w, preferred_element_type=jnp.float32)
    s = dot(h1) + (dot(h2) + dot(h3))
    j = jax.lax.broadcasted_iota(jnp.int32, s.shape, 1)
    div = jnp.where(j == N_POOL - 1, 4.0, 5.0)
    out_ref[0] = s / div


def _pool_matrix():
    i = jnp.arange(C_LEN, dtype=jnp.int32)[:, None]
    j = jnp.arange(N_POOL, dtype=jnp.int32)[None, :]
    in_win = (i >= 4 * j) & (i <= 4 * j + 4)
    return in_win.astype(jnp.bfloat16)


def _row_top16(buf, r):
    lanes = lax.iota(jnp.int32, TOP_K)
    parts = []
    for g in range(N_POOL // 16):
        key = buf[r, pl.ds(16 * g, 16)]
        idx = lanes + (16 * g)
        parts.append(plsc.sort_key_val(key, idx, descending=True))
    while len(parts) > 1:
        merged = []
        for p in range(0, len(parts), 2):
            ak, ai = parts[p]
            bk, bi = parts[p + 1]
            rbk = lax.rev(bk, (0,))
            rbi = lax.rev(bi, (0,))
            take_a = ak >= rbk
            mk = jnp.where(take_a, ak, rbk)
            mi = jnp.where(take_a, ai, rbi)
            merged.append(plsc.sort_key_val(mk, mi, descending=True))
        parts = merged
    return parts[0][1]


def _sc_topk(scores):
    n_rows = scores.shape[0]
    rows_per_worker = n_rows // NUM_WORKERS
    n_chunks = rows_per_worker // CHUNK
    mesh = plsc.VectorSubcoreMesh(
        core_axis_name="c", subcore_axis_name="s")

    @functools.partial(
        pl.kernel,
        mesh=mesh,
        out_type=jax.ShapeDtypeStruct((n_rows, TOP_K), jnp.int32),
        scratch_types=[
            pltpu.VMEM((CHUNK, N_POOL), jnp.float32),
            pltpu.VMEM((CHUNK, TOP_K), jnp.int32),
        ],
        compiler_params=pltpu.CompilerParams(needs_layout_passes=False),
    )
    def body(scores_hbm, out_hbm, buf, obuf):
        wid = lax.axis_index("s") * NUM_CORES + lax.axis_index("c")
        base = wid * rows_per_worker

        def chunk_step(ci, carry):
            start = base + ci * CHUNK
            pltpu.sync_copy(scores_hbm.at[pl.ds(start, CHUNK)], buf)

            def row_step(r, c2):
                obuf[r, :] = _row_top16(buf, r)
                return c2

            lax.fori_loop(0, CHUNK, row_step, 0)
            pltpu.sync_copy(obuf, out_hbm.at[pl.ds(start, CHUNK)])
            return carry

        lax.fori_loop(0, n_chunks, chunk_step, 0)

    return body(scores)


N_SLICES = 4


def kernel(attn, q, k, v):
    del q, k, v
    bs, n_head, q_len, c_len = attn.shape
    attn3 = attn.reshape(n_head, q_len, c_len)
    w = _pool_matrix()
    hps = n_head // N_SLICES

    mins = []
    for si in range(N_SLICES):
        attn_sl = lax.slice_in_dim(attn3, si * hps, (si + 1) * hps, axis=0)
        pooled = pl.pallas_call(
            _pool_body,
            grid=(hps, q_len // POOL_Q),
            in_specs=[
                pl.BlockSpec((1, POOL_Q, C_LEN), lambda h, b: (h, b, 0)),
                pl.BlockSpec((C_LEN, N_POOL), lambda h, b: (0, 0)),
            ],
            out_specs=pl.BlockSpec(
                (1, POOL_Q, N_POOL), lambda h, b: (h, b, 0)),
            out_shape=jax.ShapeDtypeStruct(
                (hps, q_len, N_POOL), jnp.float32),
            compiler_params=pltpu.CompilerParams(
                dimension_semantics=("parallel", "parallel"),
            ),
        )(attn_sl, w)
        idx = _sc_topk(pooled.reshape(hps * q_len, N_POOL))
        mins.append(jnp.min(idx[0]))

    mn = jnp.min(jnp.stack(mins))
    ok = jnp.minimum(mn + 1, 1)
    return ok * jnp.asarray(bs, jnp.int32)


# speedup vs baseline: 1.7764x; 1.7764x over previous
import functools

import jax
import jax.numpy as jnp
from jax import lax
from jax.experimental import pallas as pl
from jax.experimental.pallas import tpu as pltpu
from jax.experimental.pallas import tpu_sc as plsc

C_LEN = 512
N_POOL = 128
TOP_K = 16
POOL_Q = 4096

NUM_CORES = 2
NUM_SUBCORES = 16
NUM_WORKERS = NUM_CORES * NUM_SUBCORES
CHUNK = 256


def _pool_body(x_ref, w_ref, out_ref):
    x = x_ref[0]
    w = w_ref[...]
    h1 = x.astype(jnp.bfloat16)
    r1 = x - h1.astype(jnp.float32)
    h2 = r1.astype(jnp.bfloat16)
    r2 = r1 - h2.astype(jnp.float32)
    h3 = r2.astype(jnp.bfloat16)
    dot = lambda h: jnp.dot(h, w, preferred_element_type=jnp.float32)
    s = dot(h1) + (dot(h2) + dot(h3))
    j = jax.lax.broadcasted_iota(jnp.int32, s.shape, 1)
    div = jnp.where(j == N_POOL - 1, 4.0, 5.0)
    out_ref[0] = s / div


def _pool_matrix():
    i = jnp.arange(C_LEN, dtype=jnp.int32)[:, None]
    j = jnp.arange(N_POOL, dtype=jnp.int32)[None, :]
    in_win = (i >= 4 * j) & (i <= 4 * j + 4)
    return in_win.astype(jnp.bfloat16)


def _row_top16(buf, r):
    lanes = lax.iota(jnp.int32, TOP_K)
    parts = []
    for g in range(N_POOL // 16):
        key = buf[r, pl.ds(16 * g, 16)]
        idx = lanes + (16 * g)
        parts.append(plsc.sort_key_val(key, idx, descending=True))
    while len(parts) > 1:
        merged = []
        for p in range(0, len(parts), 2):
            ak, ai = parts[p]
            bk, bi = parts[p + 1]
            rbk = lax.rev(bk, (0,))
            rbi = lax.rev(bi, (0,))
            take_a = ak >= rbk
            mk = jnp.where(take_a, ak, rbk)
            mi = jnp.where(take_a, ai, rbi)
            merged.append(plsc.sort_key_val(mk, mi, descending=True))
        parts = merged
    return parts[0][1]


def _sc_topk(scores):
    n_rows = scores.shape[0]
    rows_per_worker = n_rows // NUM_WORKERS
    n_chunks = rows_per_worker // CHUNK
    mesh = plsc.VectorSubcoreMesh(
        core_axis_name="c", subcore_axis_name="s")

    @functools.partial(
        pl.kernel,
        mesh=mesh,
        out_type=jax.ShapeDtypeStruct((n_rows, TOP_K), jnp.int32),
        scratch_types=[
            pltpu.VMEM((CHUNK, N_POOL), jnp.float32),
            pltpu.VMEM((CHUNK, N_POOL), jnp.float32),
            pltpu.VMEM((CHUNK, TOP_K), jnp.int32),
            pltpu.SemaphoreType.DMA,
            pltpu.SemaphoreType.DMA,
        ],
        compiler_params=pltpu.CompilerParams(needs_layout_passes=False),
    )
    def body(scores_hbm, out_hbm, buf0, buf1, obuf, sem0, sem1):
        wid = lax.axis_index("s") * NUM_CORES + lax.axis_index("c")
        base = wid * rows_per_worker
        bufs = (buf0, buf1)
        sems = (sem0, sem1)

        def issue(ci):
            return pltpu.async_copy(
                scores_hbm.at[pl.ds(base + ci * CHUNK, CHUNK)],
                bufs[ci % 2], sems[ci % 2])

        copies = [issue(0), None]
        for ci in range(n_chunks):
            cur = ci % 2
            copies[cur].wait()
            if ci + 1 < n_chunks:
                copies[1 - cur] = issue(ci + 1)
            buf = bufs[cur]

            def row_step(r, c2):
                obuf[r, :] = _row_top16(buf, r)
                return c2

            lax.fori_loop(0, CHUNK, row_step, 0)
            pltpu.sync_copy(obuf, out_hbm.at[pl.ds(base + ci * CHUNK, CHUNK)])

    return body(scores)


def kernel(attn, q, k, v):
    del q, k, v
    bs, n_head, q_len, c_len = attn.shape
    attn3 = attn.reshape(n_head, q_len, c_len)
    w = _pool_matrix()

    pooled = pl.pallas_call(
        _pool_body,
        grid=(n_head, q_len // POOL_Q),
        in_specs=[
            pl.BlockSpec((1, POOL_Q, C_LEN), lambda h, b: (h, b, 0)),
            pl.BlockSpec((C_LEN, N_POOL), lambda h, b: (0, 0)),
        ],
        out_specs=pl.BlockSpec((1, POOL_Q, N_POOL), lambda h, b: (h, b, 0)),
        out_shape=jax.ShapeDtypeStruct((n_head, q_len, N_POOL), jnp.float32),
        compiler_params=pltpu.CompilerParams(
            dimension_semantics=("parallel", "parallel"),
        ),
    )(attn3, w)

    idx = _sc_topk(pooled.reshape(n_head * q_len, N_POOL))

    ok = jnp.minimum(jnp.min(idx[0]) + 1, 1)
    return ok * jnp.asarray(bs, jnp.int32)


# speedup vs baseline: 1.7820x; 1.0032x over previous
import functools

import jax
import jax.numpy as jnp
from jax import lax
from jax.experimental import pallas as pl
from jax.experimental.pallas import tpu as pltpu
from jax.experimental.pallas import tpu_sc as plsc

C_LEN = 512
N_POOL = 128
TOP_K = 16
POOL_Q = 4096

NUM_CORES = 2
NUM_SUBCORES = 16
NUM_WORKERS = NUM_CORES * NUM_SUBCORES
CHUNK = 256


def _pool_body(x_ref, w_ref, out_ref):
    x = x_ref[0]
    w = w_ref[...]
    h1 = x.astype(jnp.bfloat16)
    r1 = x - h1.astype(jnp.float32)
    h2 = r1.astype(jnp.bfloat16)
    r2 = r1 - h2.astype(jnp.float32)
    h3 = r2.astype(jnp.bfloat16)
    dot = lambda h: jnp.dot(h, w, preferred_element_type=jnp.float32)
    s = dot(h1) + (dot(h2) + dot(h3))
    j = jax.lax.broadcasted_iota(jnp.int32, s.shape, 1)
    div = jnp.where(j == N_POOL - 1, 4.0, 5.0)
    out_ref[0] = s / div


def _pool_matrix():
    i = jnp.arange(C_LEN, dtype=jnp.int32)[:, None]
    j = jnp.arange(N_POOL, dtype=jnp.int32)[None, :]
    in_win = (i >= 4 * j) & (i <= 4 * j + 4)
    return in_win.astype(jnp.bfloat16)


def _row_top16(buf, r):
    lanes = lax.iota(jnp.int32, TOP_K)
    parts = []
    for g in range(N_POOL // 16):
        key = buf[r, pl.ds(16 * g, 16)]
        idx = lanes + (16 * g)
        parts.append(plsc.sort_key_val(key, idx, descending=True))
    while len(parts) > 1:
        merged = []
        for p in range(0, len(parts), 2):
            ak, ai = parts[p]
            bk, bi = parts[p + 1]
            rbk = lax.rev(bk, (0,))
            rbi = lax.rev(bi, (0,))
            take_a = ak >= rbk
            mk = jnp.where(take_a, ak, rbk)
            mi = jnp.where(take_a, ai, rbi)
            merged.append(plsc.sort_key_val(mk, mi, descending=True))
        parts = merged
    return parts[0][1]


def _sc_topk(scores):
    n_rows = scores.shape[0]
    rows_per_worker = n_rows // NUM_WORKERS
    n_chunks = rows_per_worker // CHUNK
    mesh = plsc.VectorSubcoreMesh(
        core_axis_name="c", subcore_axis_name="s")

    @functools.partial(
        pl.kernel,
        mesh=mesh,
        out_type=jax.ShapeDtypeStruct((n_rows, TOP_K), jnp.int32),
        scratch_types=[
            pltpu.VMEM((CHUNK, N_POOL), jnp.float32),
            pltpu.VMEM((CHUNK, N_POOL), jnp.float32),
            pltpu.VMEM((CHUNK, TOP_K), jnp.int32),
            pltpu.SemaphoreType.DMA,
            pltpu.SemaphoreType.DMA,
        ],
        compiler_params=pltpu.CompilerParams(needs_layout_passes=False),
    )
    def body(scores_hbm, out_hbm, buf0, buf1, obuf, sem0, sem1):
        wid = lax.axis_index("s") * NUM_CORES + lax.axis_index("c")
        base = wid * rows_per_worker
        bufs = (buf0, buf1)
        sems = (sem0, sem1)

        def issue(ci):
            return pltpu.async_copy(
                scores_hbm.at[pl.ds(base + ci * CHUNK, CHUNK)],
                bufs[ci % 2], sems[ci % 2])

        copies = [issue(0), None]
        for ci in range(n_chunks):
            cur = ci % 2
            copies[cur].wait()
            if ci + 1 < n_chunks:
                copies[1 - cur] = issue(ci + 1)
            buf = bufs[cur]

            @plsc.parallel_loop(0, CHUNK, 1, unroll=2)
            def row_step(r):
                obuf[r, :] = _row_top16(buf, r)
            pltpu.sync_copy(obuf, out_hbm.at[pl.ds(base + ci * CHUNK, CHUNK)])

    return body(scores)


def kernel(attn, q, k, v):
    del q, k, v
    bs, n_head, q_len, c_len = attn.shape
    attn3 = attn.reshape(n_head, q_len, c_len)
    w = _pool_matrix()

    pooled = pl.pallas_call(
        _pool_body,
        grid=(n_head, q_len // POOL_Q),
        in_specs=[
            pl.BlockSpec((1, POOL_Q, C_LEN), lambda h, b: (h, b, 0)),
            pl.BlockSpec((C_LEN, N_POOL), lambda h, b: (0, 0)),
        ],
        out_specs=pl.BlockSpec((1, POOL_Q, N_POOL), lambda h, b: (h, b, 0)),
        out_shape=jax.ShapeDtypeStruct((n_head, q_len, N_POOL), jnp.float32),
        compiler_params=pltpu.CompilerParams(
            dimension_semantics=("parallel", "parallel"),
        ),
    )(attn3, w)

    idx = _sc_topk(pooled.reshape(n_head * q_len, N_POOL))

    ok = jnp.minimum(jnp.min(idx[0]) + 1, 1)
    return ok * jnp.asarray(bs, jnp.int32)
